# baseline (speedup 1.0000x reference)
import dataclasses
import functools

import jax
import jax.numpy as jnp
from jax import lax
from jax.experimental import pallas as pl
from jax.experimental.pallas import tpu as pltpu
from jax.experimental.pallas import tpu_sc as plsc

N_NODES = 10000
N_EDGES = 160000
D_FEAT = 256
D_EDGE = 16
D_HID = 512
D_OUT = 256

F32_MIN = float(jnp.finfo(jnp.float32).min)


NB = 1000


def _node_proj_kernel(x_ref, w_ref, o_ref):
    o_ref[...] = jnp.dot(
        x_ref[...].astype(jnp.bfloat16),
        w_ref[...].astype(jnp.bfloat16),
        preferred_element_type=jnp.float32,
    ).astype(jnp.bfloat16)


def _node_proj(nodes, w1a):
    n = nodes.shape[0]
    return pl.pallas_call(
        _node_proj_kernel,
        grid=(n // NB,),
        in_specs=[
            pl.BlockSpec((NB, D_FEAT), lambda i: (i, 0)),
            pl.BlockSpec((D_FEAT, D_HID), lambda i: (0, 0)),
        ],
        out_specs=pl.BlockSpec((NB, D_HID), lambda i: (i, 0)),
        out_shape=jax.ShapeDtypeStruct((n, D_HID), jnp.bfloat16),
    )(nodes, w1a)


GW = 128


def _sc_gather(h0, idx):
    e = idx.shape[0]
    d = h0.shape[1]
    mesh = plsc.VectorSubcoreMesh(core_axis_name="c", subcore_axis_name="s")
    idx2 = idx.reshape(1, e)

    @functools.partial(
        pl.kernel,
        out_type=jax.ShapeDtypeStruct((e, d), h0.dtype),
        mesh=mesh,
    )
    def k(h0_hbm, i_hbm, o_hbm):
        def body(i_vmem, o_vmem):
            pltpu.sync_copy(h0_hbm.at[i_vmem.at[0]], o_vmem)

        pltpu.emit_pipeline(
            body,
            grid=(e // GW,),
            in_specs=[pl.BlockSpec((1, GW), lambda i: (0, i))],
            out_specs=[pl.BlockSpec((GW, d), lambda i: (i, 0))],
            core_axis_name=("c", "s"),
            dimension_semantics=(pltpu.PARALLEL,),
        )(i_hbm, o_hbm)

    return k(h0, idx2)


EB = 1280


def _edge_mlp_kernel(g_ref, e_ref, w1b_ref, b1_ref, w2_ref, b2_ref, o_ref):
    e1 = jnp.dot(
        e_ref[...].astype(jnp.bfloat16),
        w1b_ref[...].astype(jnp.bfloat16),
        preferred_element_type=jnp.float32,
    )
    h = jnp.maximum(g_ref[...].astype(jnp.float32) + e1 + b1_ref[...], 0.0)
    o_ref[...] = (
        jnp.dot(
            h.astype(jnp.bfloat16),
            w2_ref[...].astype(jnp.bfloat16),
            preferred_element_type=jnp.float32,
        )
        + b2_ref[...]
    )


def _edge_mlp(g, edges, w1b, b1, w2, b2):
    e = g.shape[0]
    return pl.pallas_call(
        _edge_mlp_kernel,
        grid=(e // EB,),
        in_specs=[
            pl.BlockSpec((EB, D_HID), lambda i: (i, 0)),
            pl.BlockSpec((EB, D_EDGE), lambda i: (i, 0)),
            pl.BlockSpec((D_EDGE, D_HID), lambda i: (0, 0)),
            pl.BlockSpec((1, D_HID), lambda i: (0, 0)),
            pl.BlockSpec((D_HID, D_OUT), lambda i: (0, 0)),
            pl.BlockSpec((1, D_OUT), lambda i: (0, 0)),
        ],
        out_specs=pl.BlockSpec((EB, D_OUT), lambda i: (i, 0)),
        out_shape=jax.ShapeDtypeStruct((e, D_OUT), jnp.float32),
    )(g, edges, w1b, b1, w2, b2)


NW = 32
NPW = 320
CH = 6400
NCHUNK = N_EDGES // CH
MB = 64
TRASH = NPW


def _sc_cp():
    cp = pltpu.CompilerParams()
    if "needs_layout_passes" in pltpu.CompilerParams.__dataclass_fields__:
        cp = dataclasses.replace(cp, needs_layout_passes=False)
    return cp


def _sc_scatter_max(vals, receivers):
    mesh = plsc.VectorSubcoreMesh(core_axis_name="c", subcore_axis_name="s")

    @functools.partial(
        pl.kernel,
        out_type=jax.ShapeDtypeStruct((NW * NPW, D_OUT), jnp.float32),
        mesh=mesh,
        compiler_params=_sc_cp(),
        scratch_types=[
            pltpu.VMEM((NPW + 16, D_OUT), jnp.float32),
            pltpu.VMEM((CH,), jnp.int32),
            pltpu.VMEM((CH + MB,), jnp.int32),
            pltpu.VMEM((CH + MB,), jnp.int32),
            pltpu.VMEM((MB, D_OUT), jnp.float32),
        ],
    )
    def k(v_hbm, r_hbm, o_hbm, acc, rch, eid, rloc, rows):
        wid = lax.axis_index("s") * 2 + lax.axis_index("c")
        lo = wid * NPW
        neg = jnp.full((16,), F32_MIN, jnp.float32)

        @pl.loop(0, NPW + 16)
        def _(i):
            for c in range(D_OUT // 16):
                acc[i, pl.ds(c * 16, 16)] = neg

        lanes = lax.iota(jnp.int32, 16)
        dummy_e = jnp.full((16,), wid, jnp.int32)
        dummy_r = jnp.full((16,), TRASH, jnp.int32)

        @pl.loop(0, NCHUNK)
        def _(kc):
            pltpu.sync_copy(r_hbm.at[pl.ds(kc * CH, CH)], rch)

            def fbody(g, cnt):
                r = rch[pl.ds(g * 16, 16)]
                m = (r >= lo) & (r < lo + NPW)
                eidv = kc * CH + g * 16 + lanes
                plsc.store_compressed(eid.at[pl.ds(cnt, 16)], eidv, mask=m)
                plsc.store_compressed(rloc.at[pl.ds(cnt, 16)], r - lo, mask=m)
                return cnt + jnp.max(plsc.all_reduce_population_count(m))

            cnt = lax.fori_loop(0, CH // 16, fbody, jnp.int32(0))

            for j in range(MB // 16):
                eid[pl.ds(cnt + j * 16, 16)] = dummy_e
                rloc[pl.ds(cnt + j * 16, 16)] = dummy_r

            nb = (cnt + MB - 1) // MB

            def gbody(b, carry):
                pltpu.sync_copy(v_hbm.at[eid.at[pl.ds(b * MB, MB)]], rows)

                def mbody(q, c2):
                    rv = rloc[pl.ds(b * MB + q * 16, 16)]
                    for jj in range(16):
                        ro = rv[jj]
                        j = q * 16 + jj
                        for c in range(D_OUT // 16):
                            sl = pl.ds(c * 16, 16)
                            acc[ro, sl] = jnp.maximum(acc[ro, sl], rows[j, sl])
                    return c2

                return lax.fori_loop(0, MB // 16, mbody, carry)

            lax.fori_loop(0, nb, gbody, jnp.int32(0))

        pltpu.sync_copy(acc.at[pl.ds(0, NPW)], o_hbm.at[pl.ds(lo, NPW)])

    return k(vals, receivers)


def kernel(nodes, edges, senders, receivers, W1, b1, W2, b2):
    w1a = W1[:D_FEAT]
    w1b = W1[D_FEAT:]
    h0 = _node_proj(nodes, w1a)
    h0_i32 = jax.lax.bitcast_convert_type(
        h0.reshape(N_NODES, D_HID // 2, 2), jnp.int32
    )
    e_pad = 163840
    senders_p = jnp.pad(senders, (0, e_pad - N_EDGES))
    g_i32 = _sc_gather(h0_i32, senders_p)[:N_EDGES]
    g = jax.lax.bitcast_convert_type(g_i32, jnp.bfloat16).reshape(
        N_EDGES, D_HID
    )
    ne = _edge_mlp(g, edges, w1b, b1.reshape(1, -1), W2, b2.reshape(1, -1))
    out = _sc_scatter_max(ne, receivers)
    return out[:N_NODES]


# speedup vs baseline: 2.8686x; 2.8686x over previous
import dataclasses
import functools

import jax
import jax.numpy as jnp
from jax import lax
from jax.experimental import pallas as pl
from jax.experimental.pallas import tpu as pltpu
from jax.experimental.pallas import tpu_sc as plsc

N_NODES = 10000
N_EDGES = 160000
E_PAD = 163840
D_FEAT = 256
D_EDGE = 16
D_HID = 512
D_OUT = 256

F32_MIN = float(jnp.finfo(jnp.float32).min)
BF16_MIN = float(jnp.finfo(jnp.bfloat16).min)
PACKED_MIN = -8388737


def _pack16(lo_bf16, hi_bf16):
    lo = jax.lax.bitcast_convert_type(lo_bf16, jnp.uint16).astype(jnp.uint32)
    hi = jax.lax.bitcast_convert_type(hi_bf16, jnp.uint16).astype(jnp.uint32)
    return jax.lax.bitcast_convert_type(lo | (hi << 16), jnp.int32)


def _unpack16(packed_i32):
    u = jax.lax.bitcast_convert_type(packed_i32, jnp.uint32)
    lo = jax.lax.bitcast_convert_type((u & 0xFFFF).astype(jnp.uint16), jnp.bfloat16)
    hi = jax.lax.bitcast_convert_type((u >> 16).astype(jnp.uint16), jnp.bfloat16)
    return lo, hi


NB = 1000


def _node_proj_kernel(x_ref, w_ref, o_ref):
    xb = x_ref[...].astype(jnp.bfloat16)
    wb = w_ref[...].astype(jnp.bfloat16)
    h_lo = jnp.dot(xb, wb[:, : D_HID // 2], preferred_element_type=jnp.float32)
    h_hi = jnp.dot(xb, wb[:, D_HID // 2 :], preferred_element_type=jnp.float32)
    o_ref[...] = _pack16(h_lo.astype(jnp.bfloat16), h_hi.astype(jnp.bfloat16))


def _node_proj(nodes, w1a):
    n = nodes.shape[0]
    return pl.pallas_call(
        _node_proj_kernel,
        grid=(n // NB,),
        in_specs=[
            pl.BlockSpec((NB, D_FEAT), lambda i: (i, 0)),
            pl.BlockSpec((D_FEAT, D_HID), lambda i: (0, 0)),
        ],
        out_specs=pl.BlockSpec((NB, D_HID // 2), lambda i: (i, 0)),
        out_shape=jax.ShapeDtypeStruct((n, D_HID // 2), jnp.int32),
    )(nodes, w1a)


GW = 128


def _sc_gather(h0p, idx):
    e = idx.shape[0]
    d = h0p.shape[1]
    mesh = plsc.VectorSubcoreMesh(core_axis_name="c", subcore_axis_name="s")
    idx2 = idx.reshape(1, e)

    @functools.partial(
        pl.kernel,
        out_type=jax.ShapeDtypeStruct((e, d), h0p.dtype),
        mesh=mesh,
    )
    def k(h0_hbm, i_hbm, o_hbm):
        def body(i_vmem, o_vmem):
            pltpu.sync_copy(h0_hbm.at[i_vmem.at[0]], o_vmem)

        pltpu.emit_pipeline(
            body,
            grid=(e // GW,),
            in_specs=[pl.BlockSpec((1, GW), lambda i: (0, i))],
            out_specs=[pl.BlockSpec((GW, d), lambda i: (i, 0))],
            core_axis_name=("c", "s"),
            dimension_semantics=(pltpu.PARALLEL,),
        )(i_hbm, o_hbm)

    return k(h0p, idx2)


EB = 1280


def _edge_mlp_kernel(g_ref, e_ref, w1b_ref, b1_ref, w2_ref, b2_ref, o_ref):
    g_lo, g_hi = _unpack16(g_ref[...])
    e1 = jnp.dot(
        e_ref[...].astype(jnp.bfloat16),
        w1b_ref[...].astype(jnp.bfloat16),
        preferred_element_type=jnp.float32,
    )
    b1 = b1_ref[...]
    h1 = jnp.maximum(g_lo.astype(jnp.float32) + e1[:, : D_HID // 2] + b1[:, : D_HID // 2], 0.0)
    h2 = jnp.maximum(g_hi.astype(jnp.float32) + e1[:, D_HID // 2 :] + b1[:, D_HID // 2 :], 0.0)
    w2 = w2_ref[...].astype(jnp.bfloat16)
    out = (
        jnp.dot(h1.astype(jnp.bfloat16), w2[: D_HID // 2], preferred_element_type=jnp.float32)
        + jnp.dot(h2.astype(jnp.bfloat16), w2[D_HID // 2 :], preferred_element_type=jnp.float32)
        + b2_ref[...]
    )
    o_ref[...] = _pack16(
        out[:, : D_OUT // 2].astype(jnp.bfloat16),
        out[:, D_OUT // 2 :].astype(jnp.bfloat16),
    )


def _edge_mlp(g, edges, w1b, b1, w2, b2):
    return pl.pallas_call(
        _edge_mlp_kernel,
        grid=(N_EDGES // EB,),
        in_specs=[
            pl.BlockSpec((EB, D_HID // 2), lambda i: (i, 0)),
            pl.BlockSpec((EB, D_EDGE), lambda i: (i, 0)),
            pl.BlockSpec((D_EDGE, D_HID), lambda i: (0, 0)),
            pl.BlockSpec((1, D_HID), lambda i: (0, 0)),
            pl.BlockSpec((D_HID, D_OUT), lambda i: (0, 0)),
            pl.BlockSpec((1, D_OUT), lambda i: (0, 0)),
        ],
        out_specs=pl.BlockSpec((EB, D_OUT // 2), lambda i: (i, 0)),
        out_shape=jax.ShapeDtypeStruct((N_EDGES, D_OUT // 2), jnp.int32),
    )(g, edges, w1b, b1, w2, b2)


NW = 32
NPW = 320
CH = 6400
NCHUNK = N_EDGES // CH
MB = 256
TRASH = NPW
DW = D_OUT // 2


def _sc_cp():
    cp = pltpu.CompilerParams()
    if "needs_layout_passes" in pltpu.CompilerParams.__dataclass_fields__:
        cp = dataclasses.replace(cp, needs_layout_passes=False)
    return cp


def _sc_scatter_max(vals, receivers):
    mesh = plsc.VectorSubcoreMesh(core_axis_name="c", subcore_axis_name="s")

    @functools.partial(
        pl.kernel,
        out_type=jax.ShapeDtypeStruct((NW * NPW, DW), jnp.int32),
        mesh=mesh,
        compiler_params=_sc_cp(),
        scratch_types=[
            pltpu.VMEM((NPW + 16, DW), jnp.int32),
            pltpu.VMEM((CH,), jnp.int32),
            pltpu.VMEM((CH + MB + 16,), jnp.int32),
            pltpu.VMEM((CH + MB + 16,), jnp.int32),
            pltpu.VMEM((MB, DW), jnp.int32),
        ],
    )
    def k(v_hbm, r_hbm, o_hbm, acc, rch, eid, rloc, rows):
        wid = lax.axis_index("s") * 2 + lax.axis_index("c")
        lo = wid * NPW
        neg = jnp.full((16,), PACKED_MIN, jnp.int32)

        @pl.loop(0, NPW + 16)
        def _(i):
            for c in range(DW // 16):
                acc[i, pl.ds(c * 16, 16)] = neg

        lanes = lax.iota(jnp.int32, 16)
        dummy_e = jnp.full((16,), wid, jnp.int32)
        dummy_r = jnp.full((16,), TRASH, jnp.int32)

        @pl.loop(0, NCHUNK)
        def _(kc):
            pltpu.sync_copy(r_hbm.at[pl.ds(kc * CH, CH)], rch)

            def fbody(g, cnt):
                r = rch[pl.ds(g * 16, 16)]
                m = (r >= lo) & (r < lo + NPW)
                eidv = kc * CH + g * 16 + lanes
                plsc.store_compressed(eid.at[pl.ds(cnt, 16)], eidv, mask=m)
                plsc.store_compressed(rloc.at[pl.ds(cnt, 16)], r - lo, mask=m)
                return cnt + jnp.max(plsc.all_reduce_population_count(m))

            cnt = lax.fori_loop(0, CH // 16, fbody, jnp.int32(0))

            for j in range(MB // 16):
                eid[pl.ds(cnt + j * 16, 16)] = dummy_e
                rloc[pl.ds(cnt + j * 16, 16)] = dummy_r

            nb = (cnt + MB - 1) // MB

            def gbody(b, carry):
                pltpu.sync_copy(v_hbm.at[eid.at[pl.ds(b * MB, MB)]], rows)

                def mbody(q, c2):
                    rv = rloc[pl.ds(b * MB + q * 16, 16)]
                    for jj in range(16):
                        ro = rv[jj]
                        j = q * 16 + jj
                        for c in range(DW // 16):
                            sl = pl.ds(c * 16, 16)
                            a = plsc.bitcast(acc[ro, sl], jnp.bfloat16)
                            v = plsc.bitcast(rows[j, sl], jnp.bfloat16)
                            acc[ro, sl] = plsc.bitcast(
                                jnp.maximum(a, v), jnp.int32
                            )
                    return c2

                return lax.fori_loop(0, MB // 16, mbody, carry)

            lax.fori_loop(0, nb, gbody, jnp.int32(0))

        pltpu.sync_copy(acc.at[pl.ds(0, NPW)], o_hbm.at[pl.ds(lo, NPW)])

    return k(vals, receivers)


FB = 1024


def _final_kernel(x_ref, o_ref):
    lo, hi = _unpack16(x_ref[...])
    x = jnp.concatenate([lo, hi], axis=1)
    o_ref[...] = jnp.where(x == jnp.bfloat16(BF16_MIN), F32_MIN, x.astype(jnp.float32))


def _final_fix(accp):
    return pl.pallas_call(
        _final_kernel,
        grid=(NW * NPW // FB,),
        in_specs=[pl.BlockSpec((FB, DW), lambda i: (i, 0))],
        out_specs=pl.BlockSpec((FB, D_OUT), lambda i: (i, 0)),
        out_shape=jax.ShapeDtypeStruct((N_NODES, D_OUT), jnp.float32),
    )(accp)


def kernel(nodes, edges, senders, receivers, W1, b1, W2, b2):
    w1a = W1[:D_FEAT]
    w1b = W1[D_FEAT:]
    h0p = _node_proj(nodes, w1a)
    pad_idx = (jnp.arange(E_PAD - N_EDGES, dtype=jnp.int32) * 37) % N_NODES
    senders_p = jnp.concatenate([senders, pad_idx])
    gp = _sc_gather(h0p, senders_p)
    ne = _edge_mlp(gp, edges, w1b, b1.reshape(1, -1), W2, b2.reshape(1, -1))
    accp = _sc_scatter_max(ne, receivers)
    return _final_fix(accp)


# speedup vs baseline: 3.7929x; 1.3222x over previous
import dataclasses
import functools

import jax
import jax.numpy as jnp
from jax import lax
from jax.experimental import pallas as pl
from jax.experimental.pallas import tpu as pltpu
from jax.experimental.pallas import tpu_sc as plsc

N_NODES = 10000
N_EDGES = 160000
E_PAD = 163840
D_FEAT = 256
D_EDGE = 16
D_HID = 512
D_OUT = 256

F32_MIN = float(jnp.finfo(jnp.float32).min)
BF16_MIN = float(jnp.finfo(jnp.bfloat16).min)
PACKED_MIN = -8388737


def _pack16(lo_bf16, hi_bf16):
    lo = jax.lax.bitcast_convert_type(lo_bf16, jnp.uint16).astype(jnp.uint32)
    hi = jax.lax.bitcast_convert_type(hi_bf16, jnp.uint16).astype(jnp.uint32)
    return jax.lax.bitcast_convert_type(lo | (hi << 16), jnp.int32)


def _unpack16(packed_i32):
    u = jax.lax.bitcast_convert_type(packed_i32, jnp.uint32)
    lo = jax.lax.bitcast_convert_type((u & 0xFFFF).astype(jnp.uint16), jnp.bfloat16)
    hi = jax.lax.bitcast_convert_type((u >> 16).astype(jnp.uint16), jnp.bfloat16)
    return lo, hi


NB = 1000


def _node_proj_kernel(x_ref, w_ref, o_ref):
    xb = x_ref[...].astype(jnp.bfloat16)
    wb = w_ref[...].astype(jnp.bfloat16)
    h_lo = jnp.dot(xb, wb[:, : D_HID // 2], preferred_element_type=jnp.float32)
    h_hi = jnp.dot(xb, wb[:, D_HID // 2 :], preferred_element_type=jnp.float32)
    o_ref[...] = _pack16(h_lo.astype(jnp.bfloat16), h_hi.astype(jnp.bfloat16))


def _node_proj(nodes, w1a):
    n = nodes.shape[0]
    return pl.pallas_call(
        _node_proj_kernel,
        grid=(n // NB,),
        in_specs=[
            pl.BlockSpec((NB, D_FEAT), lambda i: (i, 0)),
            pl.BlockSpec((D_FEAT, D_HID), lambda i: (0, 0)),
        ],
        out_specs=pl.BlockSpec((NB, D_HID // 2), lambda i: (i, 0)),
        out_shape=jax.ShapeDtypeStruct((n, D_HID // 2), jnp.int32),
    )(nodes, w1a)


GW = 128


def _sc_gather(h0p, idx):
    e = idx.shape[0]
    d = h0p.shape[1]
    mesh = plsc.VectorSubcoreMesh(core_axis_name="c", subcore_axis_name="s")
    idx2 = idx.reshape(1, e)

    @functools.partial(
        pl.kernel,
        out_type=jax.ShapeDtypeStruct((e, d), h0p.dtype),
        mesh=mesh,
    )
    def k(h0_hbm, i_hbm, o_hbm):
        def body(i_vmem, o_vmem):
            pltpu.sync_copy(h0_hbm.at[i_vmem.at[0]], o_vmem)

        pltpu.emit_pipeline(
            body,
            grid=(e // GW,),
            in_specs=[pl.BlockSpec((1, GW), lambda i: (0, i))],
            out_specs=[pl.BlockSpec((GW, d), lambda i: (i, 0))],
            core_axis_name=("c", "s"),
            dimension_semantics=(pltpu.PARALLEL,),
        )(i_hbm, o_hbm)

    return k(h0p, idx2)


EB = 1280


def _edge_mlp_kernel(g_ref, e_ref, w1b_ref, b1_ref, w2_ref, b2_ref, o_ref):
    g_lo, g_hi = _unpack16(g_ref[...])
    e1 = jnp.dot(
        e_ref[...].astype(jnp.bfloat16),
        w1b_ref[...].astype(jnp.bfloat16),
        preferred_element_type=jnp.float32,
    )
    b1 = b1_ref[...]
    h1 = jnp.maximum(g_lo.astype(jnp.float32) + e1[:, : D_HID // 2] + b1[:, : D_HID // 2], 0.0)
    h2 = jnp.maximum(g_hi.astype(jnp.float32) + e1[:, D_HID // 2 :] + b1[:, D_HID // 2 :], 0.0)
    w2 = w2_ref[...].astype(jnp.bfloat16)
    out = (
        jnp.dot(h1.astype(jnp.bfloat16), w2[: D_HID // 2], preferred_element_type=jnp.float32)
        + jnp.dot(h2.astype(jnp.bfloat16), w2[D_HID // 2 :], preferred_element_type=jnp.float32)
        + b2_ref[...]
    )
    o_ref[...] = _pack16(
        out[:, : D_OUT // 2].astype(jnp.bfloat16),
        out[:, D_OUT // 2 :].astype(jnp.bfloat16),
    )


def _edge_mlp(g, edges, w1b, b1, w2, b2):
    return pl.pallas_call(
        _edge_mlp_kernel,
        grid=(N_EDGES // EB,),
        in_specs=[
            pl.BlockSpec((EB, D_HID // 2), lambda i: (i, 0)),
            pl.BlockSpec((EB, D_EDGE), lambda i: (i, 0)),
            pl.BlockSpec((D_EDGE, D_HID), lambda i: (0, 0)),
            pl.BlockSpec((1, D_HID), lambda i: (0, 0)),
            pl.BlockSpec((D_HID, D_OUT), lambda i: (0, 0)),
            pl.BlockSpec((1, D_OUT), lambda i: (0, 0)),
        ],
        out_specs=pl.BlockSpec((EB, D_OUT // 2), lambda i: (i, 0)),
        out_shape=jax.ShapeDtypeStruct((N_EDGES, D_OUT // 2), jnp.int32),
    )(g, edges, w1b, b1, w2, b2)


NW = 32
NPW = 320
CH = 8000
NCHUNK = N_EDGES // CH
MB = 128
TRASH = NPW
DW = D_OUT // 2


def _sc_cp():
    cp = pltpu.CompilerParams()
    if "needs_layout_passes" in pltpu.CompilerParams.__dataclass_fields__:
        cp = dataclasses.replace(cp, needs_layout_passes=False)
    return cp


def _sc_scatter_max(vals, receivers):
    mesh = plsc.VectorSubcoreMesh(core_axis_name="c", subcore_axis_name="s")

    @functools.partial(
        pl.kernel,
        out_type=jax.ShapeDtypeStruct((NW * NPW, DW), jnp.int32),
        mesh=mesh,
        compiler_params=_sc_cp(),
        scratch_types=[
            pltpu.VMEM((NPW + 16, DW), jnp.int32),
            pltpu.VMEM((CH,), jnp.int32),
            pltpu.VMEM((CH,), jnp.int32),
            pltpu.VMEM((CH + MB + 16,), jnp.int32),
            pltpu.VMEM((CH + MB + 16,), jnp.int32),
            pltpu.VMEM((MB, DW), jnp.int32),
            pltpu.VMEM((MB, DW), jnp.int32),
            pltpu.SemaphoreType.DMA,
            pltpu.SemaphoreType.DMA,
            pltpu.SemaphoreType.DMA,
        ],
    )
    def k(v_hbm, r_hbm, o_hbm, acc, rch0, rch1, eid, rloc, rows0, rows1,
          sem0, sem1, semr):
        wid = lax.axis_index("s") * 2 + lax.axis_index("c")
        lo = wid * NPW
        neg = jnp.full((16,), PACKED_MIN, jnp.int32)

        @pl.loop(0, NPW + 16)
        def _(i):
            for c in range(DW // 16):
                acc[i, pl.ds(c * 16, 16)] = neg

        lanes = lax.iota(jnp.int32, 16)
        dummy_e = wid + lanes * 512
        dummy_r = jnp.full((16,), TRASH, jnp.int32)

        def issue_gather(b, rows_ref, sem):
            pltpu.async_copy(v_hbm.at[eid.at[pl.ds(b * MB, MB)]], rows_ref, sem)

        def wait_gather(rows_ref, sem):
            pltpu.make_async_copy(v_hbm.at[eid.at[pl.ds(0, MB)]], rows_ref, sem).wait()

        def process(b, rows_ref):
            def mbody(q, c2):
                rv = rloc[pl.ds(b * MB + q * 16, 16)]
                for jj in range(16):
                    ro = rv[jj]
                    j = q * 16 + jj
                    for c in range(DW // 16):
                        sl = pl.ds(c * 16, 16)
                        a = plsc.bitcast(acc[ro, sl], jnp.bfloat16)
                        v = plsc.bitcast(rows_ref[j, sl], jnp.bfloat16)
                        acc[ro, sl] = plsc.bitcast(jnp.maximum(a, v), jnp.int32)
                return c2

            lax.fori_loop(0, MB // 16, mbody, jnp.int32(0))

        def do_chunk(kc, rch):
            def fbody(g, cnt):
                r = rch[pl.ds(g * 16, 16)]
                m = (r >= lo) & (r < lo + NPW)
                eidv = kc * CH + g * 16 + lanes
                plsc.store_compressed(eid.at[pl.ds(cnt, 16)], eidv, mask=m)
                plsc.store_compressed(rloc.at[pl.ds(cnt, 16)], r - lo, mask=m)
                return cnt + jnp.max(plsc.all_reduce_population_count(m))

            cnt = lax.fori_loop(0, CH // 16, fbody, jnp.int32(0))

            for j in range(MB // 16):
                eid[pl.ds(cnt + j * 16, 16)] = dummy_e
                rloc[pl.ds(cnt + j * 16, 16)] = dummy_r

            nb = (cnt + MB - 1) // MB

            @pl.when(nb > 0)
            def _():
                issue_gather(0, rows0, sem0)

            def pbody(bb, carry):
                b0 = bb * 2
                b1 = b0 + 1

                @pl.when(b1 < nb)
                def _():
                    issue_gather(b1, rows1, sem1)

                wait_gather(rows0, sem0)
                process(b0, rows0)

                @pl.when(b0 + 2 < nb)
                def _():
                    issue_gather(b0 + 2, rows0, sem0)

                @pl.when(b1 < nb)
                def _():
                    wait_gather(rows1, sem1)
                    process(b1, rows1)

                return carry

            lax.fori_loop(0, (nb + 1) // 2, pbody, jnp.int32(0))

        def wait_rchunk(rch):
            pltpu.make_async_copy(r_hbm.at[pl.ds(0, CH)], rch, semr).wait()

        pltpu.sync_copy(r_hbm.at[pl.ds(0, CH)], rch0)

        @pl.loop(0, NCHUNK // 2)
        def _(k2):
            kc0 = k2 * 2
            kc1 = kc0 + 1
            pltpu.async_copy(r_hbm.at[pl.ds(kc1 * CH, CH)], rch1, semr)
            do_chunk(kc0, rch0)
            wait_rchunk(rch1)

            @pl.when(kc1 + 1 < NCHUNK)
            def _():
                pltpu.async_copy(r_hbm.at[pl.ds((kc1 + 1) * CH, CH)], rch0, semr)

            do_chunk(kc1, rch1)

            @pl.when(kc1 + 1 < NCHUNK)
            def _():
                wait_rchunk(rch0)

        pltpu.sync_copy(acc.at[pl.ds(0, NPW)], o_hbm.at[pl.ds(lo, NPW)])

    return k(vals, receivers)


FB = 1024


def _final_kernel(x_ref, o_ref):
    lo, hi = _unpack16(x_ref[...])
    x = jnp.concatenate([lo, hi], axis=1)
    o_ref[...] = jnp.where(x == jnp.bfloat16(BF16_MIN), F32_MIN, x.astype(jnp.float32))


def _final_fix(accp):
    return pl.pallas_call(
        _final_kernel,
        grid=(NW * NPW // FB,),
        in_specs=[pl.BlockSpec((FB, DW), lambda i: (i, 0))],
        out_specs=pl.BlockSpec((FB, D_OUT), lambda i: (i, 0)),
        out_shape=jax.ShapeDtypeStruct((N_NODES, D_OUT), jnp.float32),
    )(accp)


def kernel(nodes, edges, senders, receivers, W1, b1, W2, b2):
    w1a = W1[:D_FEAT]
    w1b = W1[D_FEAT:]
    h0p = _node_proj(nodes, w1a)
    pad_idx = (jnp.arange(E_PAD - N_EDGES, dtype=jnp.int32) * 37) % N_NODES
    senders_p = jnp.concatenate([senders, pad_idx])
    gp = _sc_gather(h0p, senders_p)
    ne = _edge_mlp(gp, edges, w1b, b1.reshape(1, -1), W2, b2.reshape(1, -1))
    accp = _sc_scatter_max(ne, receivers)
    return _final_fix(accp)


# speedup vs baseline: 4.3724x; 1.1528x over previous
import dataclasses
import functools

import jax
import jax.numpy as jnp
from jax import lax
from jax.experimental import pallas as pl
from jax.experimental.pallas import tpu as pltpu
from jax.experimental.pallas import tpu_sc as plsc

N_NODES = 10000
N_EDGES = 160000
EH = 80000
EH_PAD = 81920
D_FEAT = 256
D_EDGE = 16
D_HID = 512
D_OUT = 256

F32_MIN = float(jnp.finfo(jnp.float32).min)
BF16_MIN = float(jnp.finfo(jnp.bfloat16).min)
PACKED_MIN = -8388737


def _pack16(lo_bf16, hi_bf16):
    lo = jax.lax.bitcast_convert_type(lo_bf16, jnp.uint16).astype(jnp.uint32)
    hi = jax.lax.bitcast_convert_type(hi_bf16, jnp.uint16).astype(jnp.uint32)
    return jax.lax.bitcast_convert_type(lo | (hi << 16), jnp.int32)


def _unpack16(packed_i32):
    u = jax.lax.bitcast_convert_type(packed_i32, jnp.uint32)
    lo = jax.lax.bitcast_convert_type((u & 0xFFFF).astype(jnp.uint16), jnp.bfloat16)
    hi = jax.lax.bitcast_convert_type((u >> 16).astype(jnp.uint16), jnp.bfloat16)
    return lo, hi


NB = 1000


def _node_proj_kernel(x_ref, w_ref, o_ref):
    xb = x_ref[...].astype(jnp.bfloat16)
    wb = w_ref[...].astype(jnp.bfloat16)
    h_lo = jnp.dot(xb, wb[:, : D_HID // 2], preferred_element_type=jnp.float32)
    h_hi = jnp.dot(xb, wb[:, D_HID // 2 :], preferred_element_type=jnp.float32)
    o_ref[...] = _pack16(h_lo.astype(jnp.bfloat16), h_hi.astype(jnp.bfloat16))


def _node_proj(nodes, w1a):
    n = nodes.shape[0]
    return pl.pallas_call(
        _node_proj_kernel,
        grid=(n // NB,),
        in_specs=[
            pl.BlockSpec((NB, D_FEAT), lambda i: (i, 0)),
            pl.BlockSpec((D_FEAT, D_HID), lambda i: (0, 0)),
        ],
        out_specs=pl.BlockSpec((NB, D_HID // 2), lambda i: (i, 0)),
        out_shape=jax.ShapeDtypeStruct((n, D_HID // 2), jnp.int32),
    )(nodes, w1a)


GW = 128


def _sc_gather(h0p, idx):
    e = idx.shape[0]
    d = h0p.shape[1]
    mesh = plsc.VectorSubcoreMesh(core_axis_name="c", subcore_axis_name="s")
    idx2 = idx.reshape(1, e)

    @functools.partial(
        pl.kernel,
        out_type=jax.ShapeDtypeStruct((e, d), h0p.dtype),
        mesh=mesh,
    )
    def k(h0_hbm, i_hbm, o_hbm):
        def body(i_vmem, o_vmem):
            pltpu.sync_copy(h0_hbm.at[i_vmem.at[0]], o_vmem)

        pltpu.emit_pipeline(
            body,
            grid=(e // GW,),
            in_specs=[pl.BlockSpec((1, GW), lambda i: (0, i))],
            out_specs=[pl.BlockSpec((GW, d), lambda i: (i, 0))],
            core_axis_name=("c", "s"),
            dimension_semantics=(pltpu.PARALLEL,),
        )(i_hbm, o_hbm)

    return k(h0p, idx2)


EB = 2000


def _edge_mlp_kernel(g_ref, e_ref, w1b_ref, b1_ref, w2_ref, b2_ref, o_ref):
    g_lo, g_hi = _unpack16(g_ref[...])
    e1 = jnp.dot(
        e_ref[...].astype(jnp.bfloat16),
        w1b_ref[...].astype(jnp.bfloat16),
        preferred_element_type=jnp.float32,
    )
    b1 = b1_ref[...]
    h1 = jnp.maximum(g_lo.astype(jnp.float32) + e1[:, : D_HID // 2] + b1[:, : D_HID // 2], 0.0)
    h2 = jnp.maximum(g_hi.astype(jnp.float32) + e1[:, D_HID // 2 :] + b1[:, D_HID // 2 :], 0.0)
    w2 = w2_ref[...].astype(jnp.bfloat16)
    out = (
        jnp.dot(h1.astype(jnp.bfloat16), w2[: D_HID // 2], preferred_element_type=jnp.float32)
        + jnp.dot(h2.astype(jnp.bfloat16), w2[D_HID // 2 :], preferred_element_type=jnp.float32)
        + b2_ref[...]
    )
    o_ref[...] = _pack16(
        out[:, : D_OUT // 2].astype(jnp.bfloat16),
        out[:, D_OUT // 2 :].astype(jnp.bfloat16),
    )


def _edge_mlp(g, edges, w1b, b1, w2, b2):
    return pl.pallas_call(
        _edge_mlp_kernel,
        grid=(EH // EB,),
        in_specs=[
            pl.BlockSpec((EB, D_HID // 2), lambda i: (i, 0)),
            pl.BlockSpec((EB, D_EDGE), lambda i: (i, 0)),
            pl.BlockSpec((D_EDGE, D_HID), lambda i: (0, 0)),
            pl.BlockSpec((1, D_HID), lambda i: (0, 0)),
            pl.BlockSpec((D_HID, D_OUT), lambda i: (0, 0)),
            pl.BlockSpec((1, D_OUT), lambda i: (0, 0)),
        ],
        out_specs=pl.BlockSpec((EB, D_OUT // 2), lambda i: (i, 0)),
        out_shape=jax.ShapeDtypeStruct((EH, D_OUT // 2), jnp.int32),
    )(g, edges, w1b, b1, w2, b2)


NW = 32
NPW = 320
CH = 8000
NCHUNK = EH // CH
MB = 128
TRASH = NPW
DW = D_OUT // 2


def _sc_cp():
    cp = pltpu.CompilerParams()
    if "needs_layout_passes" in pltpu.CompilerParams.__dataclass_fields__:
        cp = dataclasses.replace(cp, needs_layout_passes=False)
    return cp


def _sc_scatter_max(vals, receivers):
    mesh = plsc.VectorSubcoreMesh(core_axis_name="c", subcore_axis_name="s")

    @functools.partial(
        pl.kernel,
        out_type=jax.ShapeDtypeStruct((NW * NPW, DW), jnp.int32),
        mesh=mesh,
        compiler_params=_sc_cp(),
        scratch_types=[
            pltpu.VMEM((NPW + 16, DW), jnp.int32),
            pltpu.VMEM((CH,), jnp.int32),
            pltpu.VMEM((CH,), jnp.int32),
            pltpu.VMEM((CH + MB + 16,), jnp.int32),
            pltpu.VMEM((CH + MB + 16,), jnp.int32),
            pltpu.VMEM((MB, DW), jnp.int32),
            pltpu.VMEM((MB, DW), jnp.int32),
            pltpu.SemaphoreType.DMA,
            pltpu.SemaphoreType.DMA,
            pltpu.SemaphoreType.DMA,
        ],
    )
    def k(v_hbm, r_hbm, o_hbm, acc, rch0, rch1, eid, rloc, rows0, rows1,
          sem0, sem1, semr):
        wid = lax.axis_index("s") * 2 + lax.axis_index("c")
        lo = wid * NPW
        neg = jnp.full((16,), PACKED_MIN, jnp.int32)

        @pl.loop(0, NPW + 16)
        def _(i):
            for c in range(DW // 16):
                acc[i, pl.ds(c * 16, 16)] = neg

        lanes = lax.iota(jnp.int32, 16)
        dummy_e = wid + lanes * 512
        dummy_r = jnp.full((16,), TRASH, jnp.int32)

        def issue_gather(b, rows_ref, sem):
            pltpu.async_copy(v_hbm.at[eid.at[pl.ds(b * MB, MB)]], rows_ref, sem)

        def wait_gather(rows_ref, sem):
            pltpu.make_async_copy(v_hbm.at[eid.at[pl.ds(0, MB)]], rows_ref, sem).wait()

        def process(b, rows_ref):
            def mbody(q, c2):
                rv = rloc[pl.ds(b * MB + q * 16, 16)]
                for jj in range(16):
                    ro = rv[jj]
                    j = q * 16 + jj
                    for c in range(DW // 16):
                        sl = pl.ds(c * 16, 16)
                        a = plsc.bitcast(acc[ro, sl], jnp.bfloat16)
                        v = plsc.bitcast(rows_ref[j, sl], jnp.bfloat16)
                        acc[ro, sl] = plsc.bitcast(jnp.maximum(a, v), jnp.int32)
                return c2

            lax.fori_loop(0, MB // 16, mbody, jnp.int32(0))

        def do_chunk(kc, rch):
            def fbody(g, cnt):
                r = rch[pl.ds(g * 16, 16)]
                m = (r >= lo) & (r < lo + NPW)
                eidv = kc * CH + g * 16 + lanes
                plsc.store_compressed(eid.at[pl.ds(cnt, 16)], eidv, mask=m)
                plsc.store_compressed(rloc.at[pl.ds(cnt, 16)], r - lo, mask=m)
                return cnt + jnp.max(plsc.all_reduce_population_count(m))

            cnt = lax.fori_loop(0, CH // 16, fbody, jnp.int32(0))

            for j in range(MB // 16):
                eid[pl.ds(cnt + j * 16, 16)] = dummy_e
                rloc[pl.ds(cnt + j * 16, 16)] = dummy_r

            nb = (cnt + MB - 1) // MB

            @pl.when(nb > 0)
            def _():
                issue_gather(0, rows0, sem0)

            def pbody(bb, carry):
                b0 = bb * 2
                b1 = b0 + 1

                @pl.when(b1 < nb)
                def _():
                    issue_gather(b1, rows1, sem1)

                wait_gather(rows0, sem0)
                process(b0, rows0)

                @pl.when(b0 + 2 < nb)
                def _():
                    issue_gather(b0 + 2, rows0, sem0)

                @pl.when(b1 < nb)
                def _():
                    wait_gather(rows1, sem1)
                    process(b1, rows1)

                return carry

            lax.fori_loop(0, (nb + 1) // 2, pbody, jnp.int32(0))

        def wait_rchunk(rch):
            pltpu.make_async_copy(r_hbm.at[pl.ds(0, CH)], rch, semr).wait()

        pltpu.sync_copy(r_hbm.at[pl.ds(0, CH)], rch0)

        @pl.loop(0, NCHUNK // 2)
        def _(k2):
            kc0 = k2 * 2
            kc1 = kc0 + 1
            pltpu.async_copy(r_hbm.at[pl.ds(kc1 * CH, CH)], rch1, semr)
            do_chunk(kc0, rch0)
            wait_rchunk(rch1)

            @pl.when(kc1 + 1 < NCHUNK)
            def _():
                pltpu.async_copy(r_hbm.at[pl.ds((kc1 + 1) * CH, CH)], rch0, semr)

            do_chunk(kc1, rch1)

            @pl.when(kc1 + 1 < NCHUNK)
            def _():
                wait_rchunk(rch0)

        pltpu.sync_copy(acc.at[pl.ds(0, NPW)], o_hbm.at[pl.ds(lo, NPW)])

    return k(vals, receivers)


FB = 1024


def _final_kernel(x_ref, y_ref, o_ref):
    xlo, xhi = _unpack16(x_ref[...])
    ylo, yhi = _unpack16(y_ref[...])
    lo = jnp.maximum(xlo, ylo)
    hi = jnp.maximum(xhi, yhi)
    x = jnp.concatenate([lo, hi], axis=1)
    o_ref[...] = jnp.where(x == jnp.bfloat16(BF16_MIN), F32_MIN, x.astype(jnp.float32))


def _final_fix(accp1, accp2):
    return pl.pallas_call(
        _final_kernel,
        grid=(NW * NPW // FB,),
        in_specs=[
            pl.BlockSpec((FB, DW), lambda i: (i, 0)),
            pl.BlockSpec((FB, DW), lambda i: (i, 0)),
        ],
        out_specs=pl.BlockSpec((FB, D_OUT), lambda i: (i, 0)),
        out_shape=jax.ShapeDtypeStruct((N_NODES, D_OUT), jnp.float32),
    )(accp1, accp2)


def kernel(nodes, edges, senders, receivers, W1, b1, W2, b2):
    w1a = W1[:D_FEAT]
    w1b = W1[D_FEAT:]
    b1r = b1.reshape(1, -1)
    b2r = b2.reshape(1, -1)
    h0p = _node_proj(nodes, w1a)
    pad_idx = (jnp.arange(EH_PAD - EH, dtype=jnp.int32) * 37) % N_NODES
    s1 = jnp.concatenate([senders[:EH], pad_idx])
    s2 = jnp.concatenate([senders[EH:], pad_idx])
    g1 = _sc_gather(h0p, s1)
    ne1 = _edge_mlp(g1, edges[:EH], w1b, b1r, W2, b2r)
    g2 = _sc_gather(h0p, s2)
    ne2 = _edge_mlp(g2, edges[EH:], w1b, b1r, W2, b2r)
    accp1 = _sc_scatter_max(ne1, receivers[:EH])
    accp2 = _sc_scatter_max(ne2, receivers[EH:])
    return _final_fix(accp1, accp2)
